# baseline (speedup 1.0000x reference)
import functools

import jax
import jax.numpy as jnp
from jax import lax
from jax.experimental import pallas as pl
from jax.experimental.pallas import tpu as pltpu
from jax.experimental.pallas import tpu_sc as plsc

NUM_PARTITIONS = 1000000
EMBED_DIM = 64
BATCH = 16384
ROWS_PER_TILE = 8
NUM_TILES = NUM_PARTITIONS // ROWS_PER_TILE

_INFO = plsc.get_sparse_core_info()
NC, NS, L = _INFO.num_cores, _INFO.num_subcores, _INFO.num_lanes
NW = NC * NS
B_PER_W = BATCH // NW
CH = 16
NST = B_PER_W // CH
D_REGS = EMBED_DIM // L


def _body(ids_hbm, table_hbm, bias_hbm, out_hbm,
          idx_v, tiles_v, out_v, bias_v, sem0, sem1):
    c = lax.axis_index("c")
    s = lax.axis_index("s")
    wid = s * NC + c
    base = wid * B_PER_W

    pltpu.sync_copy(ids_hbm.at[wid], idx_v)
    pltpu.sync_copy(bias_hbm, bias_v)

    bias_regs = [bias_v[pl.ds(k * L, L)] for k in range(D_REGS)]
    sems = (sem0, sem1)

    def fire(st, buf, sem):
        ivec = idx_v[pl.ds(st * CH, CH)]
        tvec = lax.bitwise_and(ivec, ~(ROWS_PER_TILE - 1))
        for i in range(CH):
            t8 = pl.multiple_of(tvec[i], ROWS_PER_TILE)
            pltpu.async_copy(
                table_hbm.at[pl.ds(t8, ROWS_PER_TILE)], tiles_v.at[buf, i],
                sem)

    def drain(buf, sem):
        for i in range(CH):
            pltpu.make_async_copy(
                table_hbm.at[pl.ds(0, ROWS_PER_TILE)], tiles_v.at[buf, i],
                sem).wait()

    def extract(st, buf):
        ivec = idx_v[pl.ds(st * CH, CH)]
        rvec = lax.bitwise_and(ivec, ROWS_PER_TILE - 1)
        for i in range(CH):
            r = rvec[i]
            for k in range(D_REGS):
                out_v[st * CH + i, pl.ds(k * L, L)] = (
                    tiles_v[buf, i, r, pl.ds(k * L, L)] + bias_regs[k])

    fire(0, 0, sems[0])

    def stage_pair(p, carry):
        s0 = p * 2

        @pl.when(s0 + 1 < NST)
        def _():
            fire(s0 + 1, 1, sems[1])
        drain(0, sems[0])
        extract(s0, 0)

        @pl.when(s0 + 2 < NST)
        def _():
            fire(s0 + 2, 0, sems[0])
        drain(1, sems[1])
        extract(s0 + 1, 1)
        return carry

    lax.fori_loop(0, NST // 2, stage_pair, 0)

    pltpu.sync_copy(out_v, out_hbm.at[pl.ds(base, B_PER_W)])


@jax.jit
def _run(ids, table, bias):
    mesh = plsc.VectorSubcoreMesh(core_axis_name="c", subcore_axis_name="s")
    f = functools.partial(
        pl.kernel,
        mesh=mesh,
        out_type=jax.ShapeDtypeStruct((BATCH, EMBED_DIM), jnp.float32),
        scratch_types=[
            pltpu.VMEM((B_PER_W,), jnp.int32),
            pltpu.VMEM((2, CH, ROWS_PER_TILE, EMBED_DIM), jnp.float32),
            pltpu.VMEM((B_PER_W, EMBED_DIM), jnp.float32),
            pltpu.VMEM((EMBED_DIM,), jnp.float32),
            pltpu.SemaphoreType.DMA,
            pltpu.SemaphoreType.DMA,
        ],
    )(_body)
    return f(ids, table, bias)


def kernel(partition_ids, table, bias):
    ids = partition_ids.astype(jnp.int32).reshape(NW, B_PER_W)
    return _run(ids, table, bias)


# speedup vs baseline: 1.4725x; 1.4725x over previous
import functools

import jax
import jax.numpy as jnp
from jax import lax
from jax.experimental import pallas as pl
from jax.experimental.pallas import tpu as pltpu
from jax.experimental.pallas import tpu_sc as plsc

NUM_PARTITIONS = 1000000
EMBED_DIM = 64
BATCH = 16384
LANES = 128

_INFO = plsc.get_sparse_core_info()
NC, NS, L = _INFO.num_cores, _INFO.num_subcores, _INFO.num_lanes
NW = NC * NS
B_PER_W = BATCH // NW
GRP = B_PER_W // L
QUAD = 4
D_REGS = EMBED_DIM // L


def _body(ids_hbm, table_hbm, bias_hbm, out_hbm,
          idx_v, slabs_v, out_v, bias_v, sem0, sem1, osem):
    c = lax.axis_index("c")
    s = lax.axis_index("s")
    wid = s * NC + c
    base = wid * B_PER_W

    pltpu.sync_copy(ids_hbm.at[wid], idx_v)
    pltpu.sync_copy(bias_hbm, bias_v)

    bias_regs = [bias_v[pl.ds(k * L, L)] for k in range(D_REGS)]
    lane_iota = lax.iota(jnp.int32, L)
    cvecs = [lane_iota + (k * L) for k in range(D_REGS)]
    sems = (sem0, sem1)

    def fire_quad(jvec, q, buf, sem):
        for i in range(QUAD):
            col = pl.multiple_of(jvec[q * QUAD + i], LANES)
            pltpu.async_copy(
                table_hbm.at[:, pl.ds(col, LANES)],
                slabs_v.at[buf, i], sem)

    def drain_quad(buf, sem):
        for i in range(QUAD):
            pltpu.make_async_copy(
                table_hbm.at[:, pl.ds(0, LANES)],
                slabs_v.at[buf, i], sem).wait()

    def extract_quad(lvec, q, buf):
        for i in range(QUAD):
            lane_splat = jnp.full((L,), lvec[q * QUAD + i], jnp.int32)
            for k in range(D_REGS):
                row = plsc.load_gather(slabs_v, [
                    jnp.full((L,), buf, jnp.int32),
                    jnp.full((L,), i, jnp.int32),
                    cvecs[k], lane_splat])
                out_v[q * QUAD + i, pl.ds(k * L, L)] = row + bias_regs[k]

    def do_group(g, carry):
        ivec = idx_v[pl.ds(g * L, L)]
        jvec = lax.bitwise_and(ivec, ~(LANES - 1))
        lvec = lax.bitwise_and(ivec, LANES - 1)

        fire_quad(jvec, 0, 0, sems[0])
        fire_quad(jvec, 1, 1, sems[1])
        drain_quad(0, sems[0])
        extract_quad(lvec, 0, 0)
        fire_quad(jvec, 2, 0, sems[0])
        drain_quad(1, sems[1])
        extract_quad(lvec, 1, 1)
        fire_quad(jvec, 3, 1, sems[1])
        drain_quad(0, sems[0])
        extract_quad(lvec, 2, 0)
        drain_quad(1, sems[1])
        extract_quad(lvec, 3, 1)

        pltpu.async_copy(
            out_v, out_hbm.at[pl.ds(base + g * L, L)], osem).wait()
        return carry

    lax.fori_loop(0, GRP, do_group, 0)


@jax.jit
def _run(ids, table_t, bias):
    mesh = plsc.VectorSubcoreMesh(core_axis_name="c", subcore_axis_name="s")
    f = functools.partial(
        pl.kernel,
        mesh=mesh,
        out_type=jax.ShapeDtypeStruct((BATCH, EMBED_DIM), jnp.float32),
        scratch_types=[
            pltpu.VMEM((B_PER_W,), jnp.int32),
            pltpu.VMEM((2, QUAD, EMBED_DIM, LANES), jnp.float32),
            pltpu.VMEM((L, EMBED_DIM), jnp.float32),
            pltpu.VMEM((EMBED_DIM,), jnp.float32),
            pltpu.SemaphoreType.DMA,
            pltpu.SemaphoreType.DMA,
            pltpu.SemaphoreType.DMA,
        ],
        compiler_params=pltpu.CompilerParams(needs_layout_passes=False),
    )(_body)
    return f(ids, table_t, bias)


def kernel(partition_ids, table, bias):
    ids = partition_ids.astype(jnp.int32).reshape(NW, B_PER_W)
    return _run(ids, table.T, bias)
